# baseline (speedup 1.0000x reference)
import functools

import jax
import jax.numpy as jnp
from jax import lax
from jax.experimental import pallas as pl
from jax.experimental.pallas import tpu as pltpu
from jax.experimental.pallas import tpu_sc as plsc

NC = 2
NS = 16
LANES = 16

NNODE = 10000
NPAD = 10240
NEDGE = 320000
HALF = 128

DEG_CHUNK = 1000
EDGE_CHUNK = 128
NPK = 4
NMSG = 2


def _sc_mesh():
    return plsc.VectorSubcoreMesh(core_axis_name="c", subcore_axis_name="s")


def _deg_body(col_hbm, ew_hbm, out_hbm, col_v, ew_v, zb, acc):
    cid = lax.axis_index("c")
    sid = lax.axis_index("s")
    slab = sid * (NPAD // NS)

    def zero(i, _):
        zb[pl.ds(i * LANES, LANES)] = jnp.zeros((LANES,), jnp.float32)
        return 0

    lax.fori_loop(0, (NPAD // NS) // LANES, zero, 0)
    pltpu.sync_copy(zb, acc.at[pl.ds(slab, NPAD // NS)])
    plsc.subcore_barrier()

    wid = sid * NC + cid
    per_tile = NEDGE // (NC * NS)

    def step(i, _):
        off = wid * per_tile + i * DEG_CHUNK
        pltpu.sync_copy(col_hbm.at[pl.ds(off, DEG_CHUNK)], col_v)
        pltpu.sync_copy(ew_hbm.at[pl.ds(off, DEG_CHUNK)], ew_v)
        pltpu.sync_copy(ew_v, acc.at[col_v], add=True)
        return 0

    lax.fori_loop(0, per_tile // DEG_CHUNK, step, 0)
    plsc.subcore_barrier()
    pltpu.sync_copy(acc.at[pl.ds(slab, NPAD // NS)],
                    out_hbm.at[cid, pl.ds(slab, NPAD // NS)])


_deg_kernel = functools.partial(
    pl.kernel,
    out_type=jax.ShapeDtypeStruct((NC, NPAD), jnp.float32),
    mesh=_sc_mesh(),
    scratch_types=[
        pltpu.VMEM((DEG_CHUNK,), jnp.int32),
        pltpu.VMEM((DEG_CHUNK,), jnp.float32),
        pltpu.VMEM((NPAD // NS,), jnp.float32),
        pltpu.VMEM_SHARED((NPAD,), jnp.float32),
    ],
)(_deg_body)


def _edge_body(y_hbm, yb_hbm, pk_hbm, out_hbm,
               pk0, pk1, pk2, pk3, mb0, mb1, mf0, mf1, acc,
               gs0, gs1, ss0, ss1, ps0, ps1, ps2, ps3):
    cid = lax.axis_index("c")
    sid = lax.axis_index("s")
    half_off = cid * NNODE
    slab = sid * 624
    tail = 15 * 624

    @pl.when(sid < NS - 1)
    def _init_main():
        pltpu.sync_copy(y_hbm.at[pl.ds(half_off + slab, 624)],
                        acc.at[pl.ds(slab, 624)])

    @pl.when(sid == NS - 1)
    def _init_tail():
        pltpu.sync_copy(y_hbm.at[pl.ds(half_off + tail, 640)],
                        acc.at[pl.ds(tail, 640)])

    plsc.subcore_barrier()

    nchunks = NEDGE // EDGE_CHUNK
    rem = nchunks % NS
    nk = jnp.where(sid < rem, nchunks // NS + 1, nchunks // NS)

    pks = (pk0, pk1, pk2, pk3)
    mbs = (mb0, mb1)
    mfs = (mf0, mf1)
    gss = (gs0, gs1)
    sss = (ss0, ss1)
    pss = (ps0, ps1, ps2, ps3)

    yb_half = yb_hbm.at[pl.ds(half_off, NNODE)]

    def chunk_off(k):
        return (sid + k * NS) * EDGE_CHUNK

    def stage_pk(jp, k):
        pltpu.async_copy(pk_hbm.at[:, pl.ds(chunk_off(k), EDGE_CHUNK)],
                         pks[jp], pss[jp])

    def fire_gather(jp, jm, k):
        pltpu.make_async_copy(pk_hbm.at[:, pl.ds(chunk_off(k), EDGE_CHUNK)],
                              pks[jp], pss[jp]).wait()
        pltpu.async_copy(yb_half.at[pks[jp].at[0]], mbs[jm], gss[jm])

    def process(jp, jm):
        pltpu.make_async_copy(yb_half.at[pks[jp].at[0]], mbs[jm], gss[jm]).wait()

        def scale(g, _):
            wv = lax.bitcast_convert_type(
                pks[jp][2, pl.ds(g * LANES, LANES)], jnp.float32)
            for l in range(LANES):
                e = g * LANES + l
                w = jnp.full((LANES,), wv[l], jnp.float32)
                for q in range(HALF // (2 * LANES)):
                    v = mbs[jm][e, pl.ds(q * LANES, LANES)]
                    a = lax.bitcast_convert_type(v << 16, jnp.float32)
                    b = lax.bitcast_convert_type(
                        v & jnp.int32(-65536), jnp.float32)
                    mfs[jm][e, pl.ds(q * 2 * LANES, LANES)] = a * w
                    mfs[jm][e, pl.ds(q * 2 * LANES + LANES, LANES)] = b * w
            return 0

        lax.fori_loop(0, EDGE_CHUNK // LANES, scale, 0)
        pltpu.async_copy(mfs[jm], acc.at[pks[jp].at[1]], sss[jm], add=True)

    def wait_scatter(jp, jm):
        pltpu.make_async_copy(mfs[jm], acc.at[pks[jp].at[1]], sss[jm]).wait()

    def slab4(t, _):
        for jj in range(NPK):
            k = t * NPK + jj

            @pl.when((k >= NPK) & (k - NPK < nk))
            def _ws():
                wait_scatter(jj, jj % NMSG)

            @pl.when(k < nk)
            def _stage():
                stage_pk(jj, k)

            @pl.when((k >= 1) & (k - 1 < nk))
            def _gf():
                fire_gather((jj + NPK - 1) % NPK, (jj + 1) % NMSG, k - 1)

            @pl.when((k >= 2) & (k - 2 < nk))
            def _proc():
                process((jj + NPK - 2) % NPK, jj % NMSG)
        return 0

    max_k = nchunks // NS + 4
    lax.fori_loop(0, max_k // NPK + 1, slab4, 0)

    plsc.subcore_barrier()

    @pl.when(sid < NS - 1)
    def _out_main():
        pltpu.sync_copy(acc.at[pl.ds(slab, 624)],
                        out_hbm.at[cid, pl.ds(slab, 624)])

    @pl.when(sid == NS - 1)
    def _out_tail():
        pltpu.sync_copy(acc.at[pl.ds(tail, 640)],
                        out_hbm.at[cid, pl.ds(tail, 640)])


_edge_kernel = functools.partial(
    pl.kernel,
    out_type=jax.ShapeDtypeStruct((NC, NNODE, HALF), jnp.float32),
    mesh=_sc_mesh(),
    compiler_params=pltpu.CompilerParams(use_tc_tiling_on_sc=False),
    scratch_types=(
        [pltpu.VMEM((3, EDGE_CHUNK), jnp.int32) for _ in range(NPK)]
        + [pltpu.VMEM((EDGE_CHUNK, HALF // 2), jnp.int32) for _ in range(NMSG)]
        + [pltpu.VMEM((EDGE_CHUNK, HALF), jnp.float32) for _ in range(NMSG)]
        + [pltpu.VMEM_SHARED((NNODE, HALF), jnp.float32)]
        + [pltpu.SemaphoreType.DMA for _ in range(2 * NMSG + NPK)]
    ),
)(_edge_body)


ROWB = 2000


def _pack_bf16_words(z):
    g = z.reshape(z.shape[0], HALF // 32, 32)
    iu = lax.bitcast_convert_type(g[:, :, :16], jnp.int32) + 0x8000
    iv = lax.bitcast_convert_type(g[:, :, 16:], jnp.int32) + 0x8000
    word = (iv & jnp.int32(-65536)) | ((iu >> 16) & jnp.int32(0xFFFF))
    return word.reshape(z.shape[0], HALF // 2)


def _tc1_body(x_ref, c_ref, cb_ref, w1_ref, di_ref, y_ref, yb_ref):
    h = jnp.dot(x_ref[...], c_ref[...], preferred_element_type=jnp.float32)
    h = jnp.maximum(h + cb_ref[0], 0.0)
    xw = jnp.dot(h, w1_ref[...], preferred_element_type=jnp.float32)
    y0 = xw[:, :HALF] * di_ref[...]
    y1 = xw[:, HALF:] * di_ref[...]
    y_ref[0, :, :] = y0
    y_ref[1, :, :] = y1
    yb_ref[0, :, :] = _pack_bf16_words(y0)
    yb_ref[1, :, :] = _pack_bf16_words(y1)


def _tc3_body(a_ref, di_ref, b_ref, w_ref, y_ref, yb_ref):
    h = jnp.concatenate([a_ref[0, :, :], a_ref[1, :, :]], axis=1)
    h = jnp.maximum(h * di_ref[...] + b_ref[...], 0.0)
    xw = jnp.dot(h, w_ref[...], preferred_element_type=jnp.float32)
    y0 = xw[:, :HALF] * di_ref[...]
    y1 = xw[:, HALF:] * di_ref[...]
    y_ref[0, :, :] = y0
    y_ref[1, :, :] = y1
    yb_ref[0, :, :] = _pack_bf16_words(y0)
    yb_ref[1, :, :] = _pack_bf16_words(y1)


def _tc4_body(a_ref, di_ref, b_ref, wl_ref, bl_ref, o_ref):
    h = jnp.concatenate([a_ref[0, :, :], a_ref[1, :, :]], axis=1)
    h = jnp.maximum(h * di_ref[...] + b_ref[...], 0.0)
    o_ref[...] = (jnp.dot(h, wl_ref[...], preferred_element_type=jnp.float32)
                  + bl_ref[...])


def _grid():
    return NNODE // ROWB


def _row_spec(width):
    return pl.BlockSpec((ROWB, width), lambda i: (i, 0))


def _full_spec(shape):
    return pl.BlockSpec(shape, lambda i: tuple(0 for _ in shape))


def _pair_spec(width=HALF):
    return pl.BlockSpec((NC, ROWB, width), lambda i: (0, i, 0))


def kernel(x, edge_index, edge_weights, conv_w, conv_b, W1, b1, W2, b2, Wl, bl):
    N, F = x.shape
    K = conv_w.shape[0]
    FC = F - K + 1
    L1 = W1.shape[1]
    L2 = W2.shape[1]
    P = Wl.shape[1]

    row = edge_index[0]
    col = edge_index[1]
    pk = jnp.stack(
        [row, col, lax.bitcast_convert_type(edge_weights, jnp.int32)])

    ii = jnp.arange(F)[:, None]
    jj = jnp.arange(FC)[None, :]
    d = ii - jj
    cmat = jnp.where((d >= 0) & (d < K),
                     conv_w[jnp.clip(d, 0, K - 1)], 0.0).astype(jnp.float32)

    degp = _deg_kernel(col, edge_weights)
    deg = degp[0, :NNODE] + degp[1, :NNODE] + 1.0
    dinv = lax.rsqrt(deg).reshape(N, 1)

    y1, yb1 = pl.pallas_call(
        _tc1_body,
        grid=(_grid(),),
        in_specs=[
            _row_spec(F),
            _full_spec((F, FC)),
            pl.BlockSpec(memory_space=pltpu.SMEM),
            _full_spec((FC, L1)),
            _row_spec(1),
        ],
        out_specs=[_pair_spec(), _pair_spec(HALF // 2)],
        out_shape=[
            jax.ShapeDtypeStruct((NC, N, HALF), jnp.float32),
            jax.ShapeDtypeStruct((NC, N, HALF // 2), jnp.int32),
        ],
    )(x, cmat, conv_b, W1, dinv)

    acc1 = _edge_kernel(y1.reshape(NC * N, HALF),
                        yb1.reshape(NC * N, HALF // 2), pk)

    y2, yb2 = pl.pallas_call(
        _tc3_body,
        grid=(_grid(),),
        in_specs=[
            _pair_spec(),
            _row_spec(1),
            _full_spec((1, L1)),
            _full_spec((L1, L2)),
        ],
        out_specs=[_pair_spec(), _pair_spec(HALF // 2)],
        out_shape=[
            jax.ShapeDtypeStruct((NC, N, HALF), jnp.float32),
            jax.ShapeDtypeStruct((NC, N, HALF // 2), jnp.int32),
        ],
    )(acc1, dinv, b1.reshape(1, L1), W2)

    acc2 = _edge_kernel(y2.reshape(NC * N, HALF),
                        yb2.reshape(NC * N, HALF // 2), pk)

    out = pl.pallas_call(
        _tc4_body,
        grid=(_grid(),),
        in_specs=[
            _pair_spec(),
            _row_spec(1),
            _full_spec((1, L2)),
            _full_spec((L2, P)),
            _full_spec((1, P)),
        ],
        out_specs=_row_spec(P),
        out_shape=jax.ShapeDtypeStruct((N, P), jnp.float32),
    )(acc2, dinv, b2.reshape(1, L2), Wl, bl.reshape(1, P))

    return out


# speedup vs baseline: 1.0024x; 1.0024x over previous
import functools

import jax
import jax.numpy as jnp
from jax import lax
from jax.experimental import pallas as pl
from jax.experimental.pallas import tpu as pltpu
from jax.experimental.pallas import tpu_sc as plsc

NC = 2
NS = 16
LANES = 16

NNODE = 10000
NPAD = 10240
NEDGE = 320000
HALF = 128

DEG_CHUNK = 1000
EDGE_CHUNK = 128
NPK = 4
NMSG = 2


def _sc_mesh():
    return plsc.VectorSubcoreMesh(core_axis_name="c", subcore_axis_name="s")


def _deg_body(col_hbm, ew_hbm, out_hbm, col_v, ew_v, zb, acc):
    cid = lax.axis_index("c")
    sid = lax.axis_index("s")
    slab = sid * (NPAD // NS)

    def zero(i, _):
        zb[pl.ds(i * LANES, LANES)] = jnp.zeros((LANES,), jnp.float32)
        return 0

    lax.fori_loop(0, (NPAD // NS) // LANES, zero, 0)
    pltpu.sync_copy(zb, acc.at[pl.ds(slab, NPAD // NS)])
    plsc.subcore_barrier()

    wid = sid * NC + cid
    per_tile = NEDGE // (NC * NS)

    def step(i, _):
        off = wid * per_tile + i * DEG_CHUNK
        pltpu.sync_copy(col_hbm.at[pl.ds(off, DEG_CHUNK)], col_v)
        pltpu.sync_copy(ew_hbm.at[pl.ds(off, DEG_CHUNK)], ew_v)
        pltpu.sync_copy(ew_v, acc.at[col_v], add=True)
        return 0

    lax.fori_loop(0, per_tile // DEG_CHUNK, step, 0)
    plsc.subcore_barrier()
    pltpu.sync_copy(acc.at[pl.ds(slab, NPAD // NS)],
                    out_hbm.at[cid, pl.ds(slab, NPAD // NS)])


_deg_kernel = functools.partial(
    pl.kernel,
    out_type=jax.ShapeDtypeStruct((NC, NPAD), jnp.float32),
    mesh=_sc_mesh(),
    scratch_types=[
        pltpu.VMEM((DEG_CHUNK,), jnp.int32),
        pltpu.VMEM((DEG_CHUNK,), jnp.float32),
        pltpu.VMEM((NPAD // NS,), jnp.float32),
        pltpu.VMEM_SHARED((NPAD,), jnp.float32),
    ],
)(_deg_body)


def _edge_body(y_hbm, yb_hbm, pk_hbm, out_hbm,
               pk0, pk1, pk2, pk3, mb0, mb1, mf0, mf1, acc,
               gs0, gs1, ss0, ss1, ps0, ps1, ps2, ps3):
    cid = lax.axis_index("c")
    sid = lax.axis_index("s")
    half_off = cid * NNODE
    slab = sid * 624
    tail = 15 * 624

    @pl.when(sid < NS - 1)
    def _init_main():
        pltpu.sync_copy(y_hbm.at[pl.ds(half_off + slab, 624)],
                        acc.at[pl.ds(slab, 624)])

    @pl.when(sid == NS - 1)
    def _init_tail():
        pltpu.sync_copy(y_hbm.at[pl.ds(half_off + tail, 640)],
                        acc.at[pl.ds(tail, 640)])

    plsc.subcore_barrier()

    nchunks = NEDGE // EDGE_CHUNK
    rem = nchunks % NS
    nk = jnp.where(sid < rem, nchunks // NS + 1, nchunks // NS)

    pks = (pk0, pk1, pk2, pk3)
    mbs = (mb0, mb1)
    mfs = (mf0, mf1)
    gss = (gs0, gs1)
    sss = (ss0, ss1)
    pss = (ps0, ps1, ps2, ps3)

    yb_half = yb_hbm.at[pl.ds(half_off, NNODE)]

    def chunk_off(k):
        return (sid + k * NS) * EDGE_CHUNK

    def stage_pk(jp, k):
        pltpu.async_copy(pk_hbm.at[sid + k * NS], pks[jp], pss[jp])

    def fire_gather(jp, jm, k):
        pltpu.make_async_copy(pk_hbm.at[sid + k * NS],
                              pks[jp], pss[jp]).wait()
        pltpu.async_copy(yb_half.at[pks[jp].at[0]], mbs[jm], gss[jm])

    def process(jp, jm):
        pltpu.make_async_copy(yb_half.at[pks[jp].at[0]], mbs[jm], gss[jm]).wait()

        def scale(g, _):
            wv = lax.bitcast_convert_type(
                pks[jp][2, pl.ds(g * LANES, LANES)], jnp.float32)
            for l in range(LANES):
                e = g * LANES + l
                w = jnp.full((LANES,), wv[l], jnp.float32)
                for q in range(HALF // (2 * LANES)):
                    v = mbs[jm][e, pl.ds(q * LANES, LANES)]
                    a = lax.bitcast_convert_type(v << 16, jnp.float32)
                    b = lax.bitcast_convert_type(
                        v & jnp.int32(-65536), jnp.float32)
                    mfs[jm][e, pl.ds(q * 2 * LANES, LANES)] = a * w
                    mfs[jm][e, pl.ds(q * 2 * LANES + LANES, LANES)] = b * w
            return 0

        lax.fori_loop(0, EDGE_CHUNK // LANES, scale, 0)
        pltpu.async_copy(mfs[jm], acc.at[pks[jp].at[1]], sss[jm], add=True)

    def wait_scatter(jp, jm):
        pltpu.make_async_copy(mfs[jm], acc.at[pks[jp].at[1]], sss[jm]).wait()

    def slab4(t, _):
        for jj in range(NPK):
            k = t * NPK + jj

            @pl.when((k >= NPK) & (k - NPK < nk))
            def _ws():
                wait_scatter(jj, jj % NMSG)

            @pl.when(k < nk)
            def _stage():
                stage_pk(jj, k)

            @pl.when((k >= 1) & (k - 1 < nk))
            def _gf():
                fire_gather((jj + NPK - 1) % NPK, (jj + 1) % NMSG, k - 1)

            @pl.when((k >= 2) & (k - 2 < nk))
            def _proc():
                process((jj + NPK - 2) % NPK, jj % NMSG)
        return 0

    max_k = nchunks // NS + 4
    lax.fori_loop(0, max_k // NPK + 1, slab4, 0)

    plsc.subcore_barrier()

    @pl.when(sid < NS - 1)
    def _out_main():
        pltpu.sync_copy(acc.at[pl.ds(slab, 624)],
                        out_hbm.at[cid, pl.ds(slab, 624)])

    @pl.when(sid == NS - 1)
    def _out_tail():
        pltpu.sync_copy(acc.at[pl.ds(tail, 640)],
                        out_hbm.at[cid, pl.ds(tail, 640)])


_edge_kernel = functools.partial(
    pl.kernel,
    out_type=jax.ShapeDtypeStruct((NC, NNODE, HALF), jnp.float32),
    mesh=_sc_mesh(),
    compiler_params=pltpu.CompilerParams(use_tc_tiling_on_sc=False),
    scratch_types=(
        [pltpu.VMEM((3, EDGE_CHUNK), jnp.int32) for _ in range(NPK)]
        + [pltpu.VMEM((EDGE_CHUNK, HALF // 2), jnp.int32) for _ in range(NMSG)]
        + [pltpu.VMEM((EDGE_CHUNK, HALF), jnp.float32) for _ in range(NMSG)]
        + [pltpu.VMEM_SHARED((NNODE, HALF), jnp.float32)]
        + [pltpu.SemaphoreType.DMA for _ in range(2 * NMSG + NPK)]
    ),
)(_edge_body)


ROWB = 2000


def _pack_bf16_words(z):
    g = z.reshape(z.shape[0], HALF // 32, 32)
    iu = lax.bitcast_convert_type(g[:, :, :16], jnp.int32) + 0x8000
    iv = lax.bitcast_convert_type(g[:, :, 16:], jnp.int32) + 0x8000
    word = (iv & jnp.int32(-65536)) | ((iu >> 16) & jnp.int32(0xFFFF))
    return word.reshape(z.shape[0], HALF // 2)


def _tc1_body(x_ref, c_ref, cb_ref, w1_ref, di_ref, y_ref, yb_ref):
    h = jnp.dot(x_ref[...], c_ref[...], preferred_element_type=jnp.float32)
    h = jnp.maximum(h + cb_ref[0], 0.0)
    xw = jnp.dot(h, w1_ref[...], preferred_element_type=jnp.float32)
    y0 = xw[:, :HALF] * di_ref[...]
    y1 = xw[:, HALF:] * di_ref[...]
    y_ref[0, :, :] = y0
    y_ref[1, :, :] = y1
    yb_ref[0, :, :] = _pack_bf16_words(y0)
    yb_ref[1, :, :] = _pack_bf16_words(y1)


def _tc3_body(a_ref, di_ref, b_ref, w_ref, y_ref, yb_ref):
    h = jnp.concatenate([a_ref[0, :, :], a_ref[1, :, :]], axis=1)
    h = jnp.maximum(h * di_ref[...] + b_ref[...], 0.0)
    xw = jnp.dot(h, w_ref[...], preferred_element_type=jnp.float32)
    y0 = xw[:, :HALF] * di_ref[...]
    y1 = xw[:, HALF:] * di_ref[...]
    y_ref[0, :, :] = y0
    y_ref[1, :, :] = y1
    yb_ref[0, :, :] = _pack_bf16_words(y0)
    yb_ref[1, :, :] = _pack_bf16_words(y1)


def _tc4_body(a_ref, di_ref, b_ref, wl_ref, bl_ref, o_ref):
    h = jnp.concatenate([a_ref[0, :, :], a_ref[1, :, :]], axis=1)
    h = jnp.maximum(h * di_ref[...] + b_ref[...], 0.0)
    o_ref[...] = (jnp.dot(h, wl_ref[...], preferred_element_type=jnp.float32)
                  + bl_ref[...])


def _grid():
    return NNODE // ROWB


def _row_spec(width):
    return pl.BlockSpec((ROWB, width), lambda i: (i, 0))


def _full_spec(shape):
    return pl.BlockSpec(shape, lambda i: tuple(0 for _ in shape))


def _pair_spec(width=HALF):
    return pl.BlockSpec((NC, ROWB, width), lambda i: (0, i, 0))


def kernel(x, edge_index, edge_weights, conv_w, conv_b, W1, b1, W2, b2, Wl, bl):
    N, F = x.shape
    K = conv_w.shape[0]
    FC = F - K + 1
    L1 = W1.shape[1]
    L2 = W2.shape[1]
    P = Wl.shape[1]

    row = edge_index[0]
    col = edge_index[1]
    pk = jnp.stack(
        [row, col, lax.bitcast_convert_type(edge_weights, jnp.int32)])
    pk = pk.reshape(3, NEDGE // EDGE_CHUNK, EDGE_CHUNK).transpose(1, 0, 2)

    ii = jnp.arange(F)[:, None]
    jj = jnp.arange(FC)[None, :]
    d = ii - jj
    cmat = jnp.where((d >= 0) & (d < K),
                     conv_w[jnp.clip(d, 0, K - 1)], 0.0).astype(jnp.float32)

    degp = _deg_kernel(col, edge_weights)
    deg = degp[0, :NNODE] + degp[1, :NNODE] + 1.0
    dinv = lax.rsqrt(deg).reshape(N, 1)

    y1, yb1 = pl.pallas_call(
        _tc1_body,
        grid=(_grid(),),
        in_specs=[
            _row_spec(F),
            _full_spec((F, FC)),
            pl.BlockSpec(memory_space=pltpu.SMEM),
            _full_spec((FC, L1)),
            _row_spec(1),
        ],
        out_specs=[_pair_spec(), _pair_spec(HALF // 2)],
        out_shape=[
            jax.ShapeDtypeStruct((NC, N, HALF), jnp.float32),
            jax.ShapeDtypeStruct((NC, N, HALF // 2), jnp.int32),
        ],
    )(x, cmat, conv_b, W1, dinv)

    acc1 = _edge_kernel(y1.reshape(NC * N, HALF),
                        yb1.reshape(NC * N, HALF // 2), pk)

    y2, yb2 = pl.pallas_call(
        _tc3_body,
        grid=(_grid(),),
        in_specs=[
            _pair_spec(),
            _row_spec(1),
            _full_spec((1, L1)),
            _full_spec((L1, L2)),
        ],
        out_specs=[_pair_spec(), _pair_spec(HALF // 2)],
        out_shape=[
            jax.ShapeDtypeStruct((NC, N, HALF), jnp.float32),
            jax.ShapeDtypeStruct((NC, N, HALF // 2), jnp.int32),
        ],
    )(acc1, dinv, b1.reshape(1, L1), W2)

    acc2 = _edge_kernel(y2.reshape(NC * N, HALF),
                        yb2.reshape(NC * N, HALF // 2), pk)

    out = pl.pallas_call(
        _tc4_body,
        grid=(_grid(),),
        in_specs=[
            _pair_spec(),
            _row_spec(1),
            _full_spec((1, L2)),
            _full_spec((L2, P)),
            _full_spec((1, P)),
        ],
        out_specs=_row_spec(P),
        out_shape=jax.ShapeDtypeStruct((N, P), jnp.float32),
    )(acc2, dinv, b2.reshape(1, L2), Wl, bl.reshape(1, P))

    return out


# speedup vs baseline: 2.0009x; 1.9960x over previous
import functools

import jax
import jax.numpy as jnp
from jax import lax
from jax.experimental import pallas as pl
from jax.experimental.pallas import tpu as pltpu
from jax.experimental.pallas import tpu_sc as plsc

NC = 2
NS = 16
LANES = 16

NNODE = 10000
NPAD = 10240
NEDGE = 320000
HALF = 128

DEG_CHUNK = 1000
EDGE_CHUNK = 128
NBUF = 3


def _sc_mesh():
    return plsc.VectorSubcoreMesh(core_axis_name="c", subcore_axis_name="s")


def _deg_body(col_hbm, ew_hbm, out_hbm, col_v, ew_v, zb, acc):
    cid = lax.axis_index("c")
    sid = lax.axis_index("s")
    slab = sid * (NPAD // NS)

    def zero(i, _):
        zb[pl.ds(i * LANES, LANES)] = jnp.zeros((LANES,), jnp.float32)
        return 0

    lax.fori_loop(0, (NPAD // NS) // LANES, zero, 0)
    pltpu.sync_copy(zb, acc.at[pl.ds(slab, NPAD // NS)])
    plsc.subcore_barrier()

    wid = sid * NC + cid
    per_tile = NEDGE // (NC * NS)

    def step(i, _):
        off = wid * per_tile + i * DEG_CHUNK
        pltpu.sync_copy(col_hbm.at[pl.ds(off, DEG_CHUNK)], col_v)
        pltpu.sync_copy(ew_hbm.at[pl.ds(off, DEG_CHUNK)], ew_v)
        pltpu.sync_copy(ew_v, acc.at[col_v], add=True)
        return 0

    lax.fori_loop(0, per_tile // DEG_CHUNK, step, 0)
    plsc.subcore_barrier()
    pltpu.sync_copy(acc.at[pl.ds(slab, NPAD // NS)],
                    out_hbm.at[cid, pl.ds(slab, NPAD // NS)])


_deg_kernel = functools.partial(
    pl.kernel,
    out_type=jax.ShapeDtypeStruct((NC, NPAD), jnp.float32),
    mesh=_sc_mesh(),
    scratch_types=[
        pltpu.VMEM((DEG_CHUNK,), jnp.int32),
        pltpu.VMEM((DEG_CHUNK,), jnp.float32),
        pltpu.VMEM((NPAD // NS,), jnp.float32),
        pltpu.VMEM_SHARED((NPAD,), jnp.float32),
    ],
)(_deg_body)


def _edge_body(y_hbm, pk_hbm, out_hbm,
               pk0, pk1, pk2, m0, m1, m2, acc,
               gs0, gs1, gs2, ss0, ss1, ss2):
    cid = lax.axis_index("c")
    sid = lax.axis_index("s")
    half_off = cid * NNODE
    slab = sid * 624
    tail = 15 * 624

    @pl.when(sid < NS - 1)
    def _init_main():
        pltpu.sync_copy(y_hbm.at[pl.ds(half_off + slab, 624)],
                        acc.at[pl.ds(slab, 624)])

    @pl.when(sid == NS - 1)
    def _init_tail():
        pltpu.sync_copy(y_hbm.at[pl.ds(half_off + tail, 640)],
                        acc.at[pl.ds(tail, 640)])

    plsc.subcore_barrier()

    nchunks = NEDGE // EDGE_CHUNK
    rem = nchunks % NS
    nk = jnp.where(sid < rem, nchunks // NS + 1, nchunks // NS)

    pks = (pk0, pk1, pk2)
    msgs = (m0, m1, m2)
    gss = (gs0, gs1, gs2)
    sss = (ss0, ss1, ss2)

    y_half = y_hbm.at[pl.ds(half_off, NNODE)]

    def fire_gather(jp, jm, k):
        pltpu.sync_copy(pk_hbm.at[sid + k * NS], pks[jp])
        pltpu.async_copy(y_half.at[pks[jp].at[0]], msgs[jm], gss[jm])

    def process(jp, jm):
        pltpu.make_async_copy(y_half.at[pks[jp].at[0]],
                              msgs[jm], gss[jm]).wait()

        def scale(g, _):
            wv = lax.bitcast_convert_type(
                pks[jp][2, pl.ds(g * LANES, LANES)], jnp.float32)
            for l in range(LANES):
                e = g * LANES + l
                w = jnp.full((LANES,), wv[l], jnp.float32)
                for q in range(HALF // LANES):
                    msgs[jm][e, pl.ds(q * LANES, LANES)] = (
                        msgs[jm][e, pl.ds(q * LANES, LANES)] * w)
            return 0

        lax.fori_loop(0, EDGE_CHUNK // LANES, scale, 0)
        pltpu.async_copy(msgs[jm], acc.at[pks[jp].at[1]], sss[jm], add=True)

    def wait_scatter(jp, jm):
        pltpu.make_async_copy(msgs[jm], acc.at[pks[jp].at[1]], sss[jm]).wait()

    def slot_block(t, _):
        for jj in range(NBUF):
            k = t * NBUF + jj

            @pl.when((k >= 4) & (k - 4 < nk))
            def _ws():
                wait_scatter((jj + 2) % NBUF, (jj + 2) % NBUF)

            @pl.when((k >= 1) & (k - 1 < nk))
            def _gf():
                fire_gather((jj + 2) % NBUF, (jj + 2) % NBUF, k - 1)

            @pl.when((k >= 2) & (k - 2 < nk))
            def _proc():
                process((jj + 1) % NBUF, (jj + 1) % NBUF)
        return 0

    max_k = nchunks // NS + 4
    lax.fori_loop(0, max_k // NBUF + 1, slot_block, 0)

    plsc.subcore_barrier()

    @pl.when(sid < NS - 1)
    def _out_main():
        pltpu.sync_copy(acc.at[pl.ds(slab, 624)],
                        out_hbm.at[cid, pl.ds(slab, 624)])

    @pl.when(sid == NS - 1)
    def _out_tail():
        pltpu.sync_copy(acc.at[pl.ds(tail, 640)],
                        out_hbm.at[cid, pl.ds(tail, 640)])


_edge_kernel = functools.partial(
    pl.kernel,
    out_type=jax.ShapeDtypeStruct((NC, NNODE, HALF), jnp.float32),
    mesh=_sc_mesh(),
    scratch_types=(
        [pltpu.VMEM((3, EDGE_CHUNK), jnp.int32) for _ in range(NBUF)]
        + [pltpu.VMEM((EDGE_CHUNK, HALF), jnp.float32) for _ in range(NBUF)]
        + [pltpu.VMEM_SHARED((NNODE, HALF), jnp.float32)]
        + [pltpu.SemaphoreType.DMA for _ in range(2 * NBUF)]
    ),
)(_edge_body)


ROWB = 2000


def _tc1_body(x_ref, c_ref, cb_ref, w1_ref, di_ref, y_ref):
    h = jnp.dot(x_ref[...], c_ref[...], preferred_element_type=jnp.float32)
    h = jnp.maximum(h + cb_ref[0], 0.0)
    xw = jnp.dot(h, w1_ref[...], preferred_element_type=jnp.float32)
    y_ref[0, :, :] = xw[:, :HALF] * di_ref[...]
    y_ref[1, :, :] = xw[:, HALF:] * di_ref[...]


def _tc3_body(a_ref, di_ref, b_ref, w_ref, y_ref):
    h = jnp.concatenate([a_ref[0, :, :], a_ref[1, :, :]], axis=1)
    h = jnp.maximum(h * di_ref[...] + b_ref[...], 0.0)
    xw = jnp.dot(h, w_ref[...], preferred_element_type=jnp.float32)
    y_ref[0, :, :] = xw[:, :HALF] * di_ref[...]
    y_ref[1, :, :] = xw[:, HALF:] * di_ref[...]


def _tc4_body(a_ref, di_ref, b_ref, wl_ref, bl_ref, o_ref):
    h = jnp.concatenate([a_ref[0, :, :], a_ref[1, :, :]], axis=1)
    h = jnp.maximum(h * di_ref[...] + b_ref[...], 0.0)
    o_ref[...] = (jnp.dot(h, wl_ref[...], preferred_element_type=jnp.float32)
                  + bl_ref[...])


def _grid():
    return NNODE // ROWB


def _row_spec(width):
    return pl.BlockSpec((ROWB, width), lambda i: (i, 0))


def _full_spec(shape):
    return pl.BlockSpec(shape, lambda i: tuple(0 for _ in shape))


def _pair_spec(width=HALF):
    return pl.BlockSpec((NC, ROWB, width), lambda i: (0, i, 0))


def kernel(x, edge_index, edge_weights, conv_w, conv_b, W1, b1, W2, b2, Wl, bl):
    N, F = x.shape
    K = conv_w.shape[0]
    FC = F - K + 1
    L1 = W1.shape[1]
    L2 = W2.shape[1]
    P = Wl.shape[1]

    row = edge_index[0]
    col = edge_index[1]
    pk = jnp.stack(
        [row, col, lax.bitcast_convert_type(edge_weights, jnp.int32)])
    pk = pk.reshape(3, NEDGE // EDGE_CHUNK, EDGE_CHUNK).transpose(1, 0, 2)

    ii = jnp.arange(F)[:, None]
    jj = jnp.arange(FC)[None, :]
    d = ii - jj
    cmat = jnp.where((d >= 0) & (d < K),
                     conv_w[jnp.clip(d, 0, K - 1)], 0.0).astype(jnp.float32)

    degp = _deg_kernel(col, edge_weights)
    deg = degp[0, :NNODE] + degp[1, :NNODE] + 1.0
    dinv = lax.rsqrt(deg).reshape(N, 1)

    y1 = pl.pallas_call(
        _tc1_body,
        grid=(_grid(),),
        in_specs=[
            _row_spec(F),
            _full_spec((F, FC)),
            pl.BlockSpec(memory_space=pltpu.SMEM),
            _full_spec((FC, L1)),
            _row_spec(1),
        ],
        out_specs=_pair_spec(),
        out_shape=jax.ShapeDtypeStruct((NC, N, HALF), jnp.float32),
    )(x, cmat, conv_b, W1, dinv)

    acc1 = _edge_kernel(y1.reshape(NC * N, HALF), pk)

    y2 = pl.pallas_call(
        _tc3_body,
        grid=(_grid(),),
        in_specs=[
            _pair_spec(),
            _row_spec(1),
            _full_spec((1, L1)),
            _full_spec((L1, L2)),
        ],
        out_specs=_pair_spec(),
        out_shape=jax.ShapeDtypeStruct((NC, N, HALF), jnp.float32),
    )(acc1, dinv, b1.reshape(1, L1), W2)

    acc2 = _edge_kernel(y2.reshape(NC * N, HALF), pk)

    out = pl.pallas_call(
        _tc4_body,
        grid=(_grid(),),
        in_specs=[
            _pair_spec(),
            _row_spec(1),
            _full_spec((1, L2)),
            _full_spec((L2, P)),
            _full_spec((1, P)),
        ],
        out_specs=_row_spec(P),
        out_shape=jax.ShapeDtypeStruct((N, P), jnp.float32),
    )(acc2, dinv, b2.reshape(1, L2), Wl, bl.reshape(1, P))

    return out
